# baseline (speedup 1.0000x reference)
import functools

import jax
import jax.numpy as jnp
from jax import lax
from jax.experimental import pallas as pl
from jax.experimental.pallas import tpu as pltpu
from jax.experimental.pallas import tpu_sc as plsc

N = 10000
E = 320000
D = 128
DP = 144
NC = 2
NS = 16
NW = NC * NS
EPW = E // NW
K = 80
GC = 1
NCHUNK = EPW // K
NG = NCHUNK // GC
RPS = N // NS

_mesh = plsc.VectorSubcoreMesh(core_axis_name="c", subcore_axis_name="s")

NR = 3
NI = 6
_row_scratch = [pltpu.VMEM((K, D), jnp.float32) for _ in range(NR)]
_idx_scratch = [pltpu.VMEM((2, K), jnp.int32) for _ in range(NI)]
_sem_scratch = [pltpu.SemaphoreType.DMA for _ in range(NR + NI)]


@functools.partial(
    pl.kernel,
    mesh=_mesh,
    out_type=(jax.ShapeDtypeStruct((NC, N, D), jnp.float32),
              jax.ShapeDtypeStruct((NC, NS, N), jnp.float32)),
    scratch_types=[pltpu.VMEM_SHARED((N, D), jnp.float32),
                   pltpu.VMEM((N,), jnp.float32)]
    + _row_scratch + _idx_scratch + _sem_scratch,
    compiler_params=pltpu.CompilerParams(use_tc_tiling_on_sc=False,
                                         needs_layout_passes=False),
)
def _sc_segment_sum(x_hbm, edge_hbm, zeros_hbm,
                    out_hbm, cnt_hbm, acc, hist, *scratch):
    rows = scratch[:NR]
    idxb = scratch[NR:NR + NI]
    sems = scratch[NR + NI:2 * NR + NI]
    isem = scratch[2 * NR + NI:]

    c = lax.axis_index("c")
    s = lax.axis_index("s")
    wid = s * NC + c
    base = wid * EPW

    def fire_idx(g, r):
        off = pl.multiple_of(base + g * K, 8)
        pltpu.async_copy(edge_hbm.at[0, pl.ds(off, K)], idxb[r].at[0], isem[r])
        pltpu.async_copy(edge_hbm.at[1, pl.ds(off, K)], idxb[r].at[1], isem[r])

    def wait_idx(r):
        for _ in range(2):
            pltpu.make_async_copy(edge_hbm.at[0, pl.ds(0, K)],
                                  idxb[r].at[0], isem[r]).wait()

    def fire_b(g, p, r):
        pltpu.async_copy(x_hbm.at[idxb[r].at[0]], rows[p], sems[p])

    one16 = jnp.full((16,), 1.0, jnp.float32)

    def fire_c(g, p, r):
        pltpu.async_copy(rows[p], acc.at[idxb[r].at[1]], sems[p], add=True)

    def hist_update(r):
        for j in range(K // 16):
            idx16 = idxb[r][1, pl.ds(16 * j, 16)]
            plsc.addupdate_scatter(hist, [idx16], one16)

    def wait_b(p):
        pltpu.make_async_copy(x_hbm.at[pl.ds(0, K)], rows[p], sems[p]).wait()

    def wait_c(p):
        pltpu.make_async_copy(rows[p], acc.at[pl.ds(0, K)], sems[p]).wait()

    zero16 = jnp.zeros((16,), jnp.float32)

    def zero_hist(i, carry):
        hist[pl.ds(pl.multiple_of(i * 16, 8), 16)] = zero16
        return carry

    lax.fori_loop(0, N // 16, zero_hist, 0)

    fire_idx(0, 0)
    fire_idx(1, 1)
    fire_idx(2, 2)
    wait_idx(0)
    fire_b(0, 0, 0)
    pltpu.sync_copy(zeros_hbm.at[pl.ds(s * RPS, RPS)],
                    acc.at[pl.ds(s * RPS, RPS)])
    plsc.subcore_barrier()

    wait_b(0)
    fire_c(0, 0, 0)
    hist_update(0)
    wait_idx(1)
    fire_b(1, 1, 1)
    fire_idx(3, 3)
    wait_b(1)
    fire_c(1, 1, 1)
    hist_update(1)
    wait_idx(2)
    fire_b(2, 2, 2)
    fire_idx(4, 4)

    def group_body(g, a, ib, last_idx, last_b):
        wait_b(a)
        fire_c(g, a, ib)
        hist_update(ib)
        if not last_b:
            wait_idx((ib + 1) % NI)
        wait_c((a + 1) % NR)
        if not last_b:
            fire_b(g + 1, (a + 1) % NR, (ib + 1) % NI)
        if not last_idx:
            fire_idx(g + 3, (ib + 3) % NI)

    def sextet(i, carry):
        g = 2 + 6 * i
        for j in range(6):
            group_body(g + j, (2 + j) % NR, (2 + j) % NI, False, False)
        return carry

    lax.fori_loop(0, (NG - 5) // 6, sextet, 0)

    group_body(NG - 3, (NG - 3) % NR, (NG - 3) % NI, True, False)
    group_body(NG - 2, (NG - 2) % NR, (NG - 2) % NI, True, False)
    group_body(NG - 1, (NG - 1) % NR, (NG - 1) % NI, True, True)
    wait_c((NG - 2) % NR)
    wait_c((NG - 1) % NR)
    plsc.subcore_barrier()

    pltpu.sync_copy(acc.at[pl.ds(s * RPS, RPS)],
                    out_hbm.at[c, pl.ds(s * RPS, RPS)])
    pltpu.sync_copy(hist, cnt_hbm.at[c, s])


BLK = 2000


def _tc_epilogue(x_ref, p_ref, q_ref, wt_ref, b_ref, o_ref):
    p = p_ref[...]
    q = q_ref[...]
    ssum = p[0] + p[1]
    cnt = jnp.sum(q, axis=1, keepdims=True)
    mean = ssum / jnp.maximum(cnt, 1.0)
    a = jnp.concatenate([x_ref[...], mean], axis=1)
    h = lax.dot_general(a, wt_ref[...], (((1,), (0,)), ((), ())),
                        preferred_element_type=jnp.float32)
    o_ref[...] = jnp.where(cnt > 0.0, h + b_ref[...], 0.0)


_epilogue_call = pl.pallas_call(
    _tc_epilogue,
    grid=(N // BLK,),
    in_specs=[
        pl.BlockSpec((BLK, D), lambda i: (i, 0)),
        pl.BlockSpec((NC, BLK, D), lambda i: (0, i, 0)),
        pl.BlockSpec((BLK, NW), lambda i: (i, 0)),
        pl.BlockSpec((2 * D, D), lambda i: (0, 0)),
        pl.BlockSpec((1, D), lambda i: (0, 0)),
    ],
    out_specs=pl.BlockSpec((BLK, D), lambda i: (i, 0)),
    out_shape=jax.ShapeDtypeStruct((N, D), jnp.float32),
)


def kernel(x, edge_index, W, b):
    zeros = jnp.zeros((N, D), jnp.float32)
    partial, hists = _sc_segment_sum(x, edge_index, zeros)
    counts = jnp.transpose(hists.reshape(NW, N))
    return _epilogue_call(x, partial, counts, W.T, b.reshape(1, D))


# speedup vs baseline: 1.0005x; 1.0005x over previous
import functools

import jax
import jax.numpy as jnp
from jax import lax
from jax.experimental import pallas as pl
from jax.experimental.pallas import tpu as pltpu
from jax.experimental.pallas import tpu_sc as plsc

N = 10000
E = 320000
D = 128
NC = 2
NS = 16
NW = NC * NS
EPW = E // NW
K = 80
NCHUNK = EPW // K
NG = NCHUNK
RPS = N // NS

_mesh = plsc.VectorSubcoreMesh(core_axis_name="c", subcore_axis_name="s")

NR = 3
NI = 6
_row_scratch = [pltpu.VMEM((K, D), jnp.float32) for _ in range(NR)]
_idx_scratch = [pltpu.VMEM((2, K), jnp.int32) for _ in range(NI)]
_sem_scratch = [pltpu.SemaphoreType.DMA for _ in range(NR + NI)]


@functools.partial(
    pl.kernel,
    mesh=_mesh,
    out_type=(jax.ShapeDtypeStruct((NC, N, D), jnp.float32),
              jax.ShapeDtypeStruct((NC, NS, N), jnp.float32)),
    scratch_types=[pltpu.VMEM_SHARED((N, D), jnp.float32),
                   pltpu.VMEM((N,), jnp.float32)]
    + _row_scratch + _idx_scratch + _sem_scratch,
    compiler_params=pltpu.CompilerParams(use_tc_tiling_on_sc=False,
                                         needs_layout_passes=False),
)
def _sc_segment_sum(x_hbm, edge_hbm, zeros_hbm,
                    out_hbm, cnt_hbm, acc, hist, *scratch):
    rows = scratch[:NR]
    idxb = scratch[NR:NR + NI]
    sems = scratch[NR + NI:2 * NR + NI]
    isem = scratch[2 * NR + NI:]

    c = lax.axis_index("c")
    s = lax.axis_index("s")
    wid = s * NC + c
    base = wid * EPW

    def fire_idx(g, r):
        off = pl.multiple_of(base + g * K, 8)
        pltpu.async_copy(edge_hbm.at[0, pl.ds(off, K)], idxb[r].at[0], isem[r])
        pltpu.async_copy(edge_hbm.at[1, pl.ds(off, K)], idxb[r].at[1], isem[r])

    def wait_idx(r):
        for _ in range(2):
            pltpu.make_async_copy(edge_hbm.at[0, pl.ds(0, K)],
                                  idxb[r].at[0], isem[r]).wait()

    def fire_b(g, p, r):
        pltpu.async_copy(x_hbm.at[idxb[r].at[0]], rows[p], sems[p])

    one16 = jnp.full((16,), 1.0, jnp.float32)

    def fire_c(g, p, r):
        pltpu.async_copy(rows[p], acc.at[idxb[r].at[1]], sems[p], add=True)

    def hist_update(r):
        for j in range(K // 16):
            idx16 = idxb[r][1, pl.ds(16 * j, 16)]
            plsc.addupdate_scatter(hist, [idx16], one16)

    def wait_b(p):
        pltpu.make_async_copy(x_hbm.at[pl.ds(0, K)], rows[p], sems[p]).wait()

    def wait_c(p):
        pltpu.make_async_copy(rows[p], acc.at[pl.ds(0, K)], sems[p]).wait()

    zero16 = jnp.zeros((16,), jnp.float32)

    def zero_hist(i, carry):
        hist[pl.ds(pl.multiple_of(i * 16, 8), 16)] = zero16
        return carry

    lax.fori_loop(0, N // 16, zero_hist, 0)

    fire_idx(0, 0)
    fire_idx(1, 1)
    fire_idx(2, 2)
    wait_idx(0)
    fire_b(0, 0, 0)
    pltpu.sync_copy(zeros_hbm.at[pl.ds(s * RPS, RPS)],
                    acc.at[pl.ds(s * RPS, RPS)])
    plsc.subcore_barrier()

    wait_b(0)
    fire_c(0, 0, 0)
    hist_update(0)
    wait_idx(1)
    fire_b(1, 1, 1)
    fire_idx(3, 3)
    wait_b(1)
    fire_c(1, 1, 1)
    hist_update(1)
    wait_idx(2)
    fire_b(2, 2, 2)
    fire_idx(4, 4)

    def group_body(g, a, ib, last_idx, last_b):
        wait_b(a)
        fire_c(g, a, ib)
        hist_update(ib)
        if not last_b:
            wait_idx((ib + 1) % NI)
        wait_c((a + 1) % NR)
        if not last_b:
            fire_b(g + 1, (a + 1) % NR, (ib + 1) % NI)
        if not last_idx:
            fire_idx(g + 3, (ib + 3) % NI)

    def sextet(i, carry):
        g = 2 + 6 * i
        for j in range(6):
            group_body(g + j, (2 + j) % NR, (2 + j) % NI, False, False)
        return carry

    lax.fori_loop(0, (NG - 5) // 6, sextet, 0)

    group_body(NG - 3, (NG - 3) % NR, (NG - 3) % NI, True, False)
    group_body(NG - 2, (NG - 2) % NR, (NG - 2) % NI, True, False)
    group_body(NG - 1, (NG - 1) % NR, (NG - 1) % NI, True, True)
    wait_c((NG - 2) % NR)
    wait_c((NG - 1) % NR)
    plsc.subcore_barrier()

    pltpu.sync_copy(acc.at[pl.ds(s * RPS, RPS)],
                    out_hbm.at[c, pl.ds(s * RPS, RPS)])
    pltpu.sync_copy(hist, cnt_hbm.at[c, s])


BLK = 2000


def _tc_epilogue(x_ref, p_ref, q_ref, wt_ref, b_ref, o_ref):
    p = p_ref[...]
    q = q_ref[...]
    ssum = p[0] + p[1]
    cnt = jnp.sum(q, axis=1, keepdims=True)
    mean = ssum / jnp.maximum(cnt, 1.0)
    a = jnp.concatenate([x_ref[...], mean], axis=1)
    h = lax.dot_general(a, wt_ref[...], (((1,), (0,)), ((), ())),
                        preferred_element_type=jnp.float32)
    o_ref[...] = jnp.where(cnt > 0.0, h + b_ref[...], 0.0)


_epilogue_call = pl.pallas_call(
    _tc_epilogue,
    grid=(N // BLK,),
    in_specs=[
        pl.BlockSpec((BLK, D), lambda i: (i, 0)),
        pl.BlockSpec((NC, BLK, D), lambda i: (0, i, 0)),
        pl.BlockSpec((BLK, NW), lambda i: (i, 0)),
        pl.BlockSpec((2 * D, D), lambda i: (0, 0)),
        pl.BlockSpec((1, D), lambda i: (0, 0)),
    ],
    out_specs=pl.BlockSpec((BLK, D), lambda i: (i, 0)),
    out_shape=jax.ShapeDtypeStruct((N, D), jnp.float32),
)


def kernel(x, edge_index, W, b):
    zeros = jnp.zeros((N, D), jnp.float32)
    partial, hists = _sc_segment_sum(x, edge_index, zeros)
    counts = jnp.transpose(hists.reshape(NW, N))
    return _epilogue_call(x, partial, counts, W.T, b.reshape(1, D))
